# Initial kernel scaffold; baseline (speedup 1.0000x reference)
import jax
import jax.numpy as jnp
from jax.experimental import pallas as pl


def kernel(x, edge_index, batch, Ws, att_src, att_dst, biases, Wd, bd, Wmu, bmu):
    raise NotImplementedError("write your pallas kernel here")


# speedup vs baseline: 26.7124x; 26.7124x over previous
import functools

import jax
import jax.numpy as jnp
from jax import lax
from jax.experimental import pallas as pl
from jax.experimental.pallas import tpu as pltpu
from jax.experimental.pallas import tpu_sc as plsc

NGRAPH = 64
NP = 10240
CH = 128
BS = 1152


def _tc_first(x, W, a_s, a_d):
    n, d = x.shape

    def body(x_ref, w_ref, as_ref, ad_ref, h_ref, sv_ref, dv_ref):
        h = jnp.dot(x_ref[...], w_ref[...], preferred_element_type=jnp.float32)
        h_ref[...] = h
        sv_ref[...] = jnp.sum(h * as_ref[...], axis=1, keepdims=True)
        dv_ref[...] = jnp.sum(h * ad_ref[...], axis=1, keepdims=True)

    return pl.pallas_call(
        body,
        out_shape=(jax.ShapeDtypeStruct((n, d), jnp.float32),
                   jax.ShapeDtypeStruct((n, 1), jnp.float32),
                   jax.ShapeDtypeStruct((n, 1), jnp.float32)),
    )(x, W, a_s, a_d)


def _tc_mid(p, b, W, a_s, a_d, n):
    d = W.shape[0]

    def body(p_ref, b_ref, w_ref, as_ref, ad_ref, h_ref, sv_ref, dv_ref):
        a = jnp.maximum(p_ref[0, :n, :] + p_ref[1, :n, :] + b_ref[...], 0.0)
        h = jnp.dot(a, w_ref[...], preferred_element_type=jnp.float32)
        h_ref[...] = h
        sv_ref[...] = jnp.sum(h * as_ref[...], axis=1, keepdims=True)
        dv_ref[...] = jnp.sum(h * ad_ref[...], axis=1, keepdims=True)

    return pl.pallas_call(
        body,
        out_shape=(jax.ShapeDtypeStruct((n, d), jnp.float32),
                   jax.ShapeDtypeStruct((n, 1), jnp.float32),
                   jax.ShapeDtypeStruct((n, 1), jnp.float32)),
    )(p, b, W, a_s, a_d)


def _tc_head(p, b, batch2d, Wd, bd, Wmu, bmu, n):
    def body(p_ref, b_ref, bt_ref, wd_ref, bd_ref, wmu_ref, bmu_ref, mu_ref):
        a = jnp.maximum(p_ref[0, :n, :] + p_ref[1, :n, :] + b_ref[...], 0.0)
        gid = lax.broadcasted_iota(jnp.int32, (NGRAPH, n), 0)
        oh = jnp.where(gid == bt_ref[...], 1.0, 0.0)
        sums = jnp.dot(oh, a, preferred_element_type=jnp.float32)
        counts = jnp.sum(oh, axis=1, keepdims=True)
        pooled = sums / jnp.maximum(counts, 1.0)
        z = jnp.maximum(
            jnp.dot(pooled, wd_ref[...], preferred_element_type=jnp.float32)
            + bd_ref[...], 0.0)
        mu_ref[...] = (jnp.dot(z, wmu_ref[...], preferred_element_type=jnp.float32)
                       + bmu_ref[...])

    return pl.pallas_call(
        body,
        out_shape=jax.ShapeDtypeStruct((NGRAPH, 1), jnp.float32),
    )(p, b, batch2d, Wd, bd, Wmu, bmu)


def _sc_layer(src1d, dst1d, h, asv, adv, n, d, ep):
    ep16 = ep // 16
    ep32 = ep // 32
    sl = NP // 16
    mesh = plsc.VectorSubcoreMesh(core_axis_name="c", subcore_axis_name="s")

    @functools.partial(
        pl.kernel,
        out_type=(jax.ShapeDtypeStruct((2, NP, d), jnp.float32),
                  jax.ShapeDtypeStruct((2, ep), jnp.float32)),
        mesh=mesh,
        compiler_params=pltpu.CompilerParams(needs_layout_passes=False),
        scratch_types=[
            pltpu.VMEM((BS,), jnp.int32),
            pltpu.VMEM((BS,), jnp.int32),
            pltpu.VMEM((BS,), jnp.float32),
            pltpu.VMEM((CH,), jnp.int32),
            pltpu.VMEM((CH,), jnp.int32),
            pltpu.VMEM((NP,), jnp.float32),
            pltpu.VMEM((NP,), jnp.float32),
            pltpu.VMEM((CH,), jnp.float32),
            pltpu.VMEM((CH, d), jnp.float32),
            pltpu.VMEM_SHARED((NP,), jnp.float32),
            pltpu.VMEM_SHARED((NP, d), jnp.float32),
            pltpu.SemaphoreType.DMA,
        ],
    )
    def kern(src_hbm, dst_hbm, h_hbm, as_hbm, ad_hbm, out_hbm, ex_hbm,
             dsti, srci, exb, dstc, srcc, as_v, ad_v, exv, rows,
             den_sp, out_sp, sem):
        c = lax.axis_index("c")
        s = lax.axis_index("s")
        wid = c * 16 + s
        z16 = jnp.zeros((16,), jnp.float32)

        pltpu.sync_copy(as_hbm, as_v.at[pl.ds(0, n)])
        pltpu.sync_copy(ad_hbm, ad_v.at[pl.ds(0, n)])
        for t in range(n, NP, 16):
            as_v[pl.ds(t, 16)] = z16
            ad_v[pl.ds(t, 16)] = z16

        def _zr(r, carry):
            for k8 in range(8):
                rows[r, pl.ds(k8 * 16, 16)] = z16
            return carry

        lax.fori_loop(0, CH, _zr, 0)
        for k8 in range(CH // 16):
            exv[pl.ds(k8 * 16, 16)] = z16
        for t in range(sl // CH):
            pltpu.sync_copy(rows, out_sp.at[pl.ds(s * sl + t * CH, CH)])
            pltpu.sync_copy(exv, den_sp.at[pl.ds(s * sl + t * CH, CH)])
        plsc.subcore_barrier()

        def _dblk(b, carry):
            base = s * ep16 + b * BS
            pltpu.sync_copy(dst_hbm.at[pl.ds(base, BS)], dsti)
            pltpu.sync_copy(src_hbm.at[pl.ds(base, BS)], srci)

            def _dch(q, c1):
                def _grp(j, c2):
                    o = q * CH + j * 16
                    si = srci[pl.ds(o, 16)]
                    di = dsti[pl.ds(o, 16)]
                    dstc[pl.ds(j * 16, 16)] = di
                    a = (plsc.load_gather(as_v, [si])
                         + plsc.load_gather(ad_v, [di]))
                    a = jnp.maximum(a, 0.2 * a)
                    e = jnp.exp(a)
                    exv[pl.ds(j * 16, 16)] = e
                    exb[pl.ds(o, 16)] = e
                    return c2

                lax.fori_loop(0, CH // 16, _grp, 0)
                pltpu.sync_copy(exv, den_sp.at[dstc], add=True)
                return c1

            lax.fori_loop(0, BS // CH, _dch, 0)
            pltpu.sync_copy(exb, ex_hbm.at[c, pl.ds(base, BS)])
            return carry

        lax.fori_loop(0, ep16 // BS, _dblk, 0)
        plsc.subcore_barrier()
        pltpu.sync_copy(den_sp, as_v)

        def _rblk(b, carry):
            base = wid * ep32 + b * BS
            pltpu.sync_copy(dst_hbm.at[pl.ds(base, BS)], dsti)
            pltpu.sync_copy(src_hbm.at[pl.ds(base, BS)], srci)
            pltpu.sync_copy(ex_hbm.at[c, pl.ds(base, BS)], exb)

            def _rch(q, c1):
                def _grp(j, c2):
                    o = q * CH + j * 16
                    si = srci[pl.ds(o, 16)]
                    di = dsti[pl.ds(o, 16)]
                    srcc[pl.ds(j * 16, 16)] = si
                    dstc[pl.ds(j * 16, 16)] = di
                    e = exb[pl.ds(o, 16)]
                    dn = plsc.load_gather(as_v, [di])
                    exv[pl.ds(j * 16, 16)] = e / (dn + 1e-16)
                    return c2

                lax.fori_loop(0, CH // 16, _grp, 0)
                pltpu.async_copy(h_hbm.at[srcc], rows, sem).wait()

                def _sgrp(g, c2):
                    cfv = exv[pl.ds(g * 16, 16)]
                    for r16 in range(16):
                        cf = cfv[r16]
                        r = g * 16 + r16
                        for k8 in range(8):
                            rows[r, pl.ds(k8 * 16, 16)] = (
                                rows[r, pl.ds(k8 * 16, 16)] * cf)
                    return c2

                lax.fori_loop(0, CH // 16, _sgrp, 0)
                pltpu.sync_copy(rows, out_sp.at[dstc], add=True)
                return c1

            lax.fori_loop(0, BS // CH, _rch, 0)
            return carry

        lax.fori_loop(0, ep32 // BS, _rblk, 0)
        plsc.subcore_barrier()
        pltpu.sync_copy(out_sp.at[pl.ds(s * sl, sl)],
                        out_hbm.at[c, pl.ds(s * sl, sl)])

    return kern(src1d, dst1d, h, asv, adv)[0]


def kernel(x, edge_index, batch, Ws, att_src, att_dst, biases, Wd, bd, Wmu, bmu):
    n, d = x.shape
    nlayers = Ws.shape[0]
    e = edge_index.shape[1]
    ep = ((e + n + 4095) // 4096) * 4096
    pad = ep - (e + n)
    loop = jnp.arange(n, dtype=jnp.int32)
    src = jnp.concatenate([edge_index[0], loop, jnp.zeros((pad,), jnp.int32)])
    dst = jnp.concatenate([edge_index[1], loop, jnp.full((pad,), n, jnp.int32)])

    h, sv, dv = _tc_first(x, Ws[0], att_src[0].reshape(1, d),
                          att_dst[0].reshape(1, d))
    p = _sc_layer(src, dst, h, sv.reshape(-1), dv.reshape(-1), n, d, ep)
    for i in range(1, nlayers):
        h, sv, dv = _tc_mid(p, biases[i - 1].reshape(1, d), Ws[i],
                            att_src[i].reshape(1, d), att_dst[i].reshape(1, d), n)
        p = _sc_layer(src, dst, h, sv.reshape(-1), dv.reshape(-1), n, d, ep)
    mu = _tc_head(p, biases[nlayers - 1].reshape(1, d), batch.reshape(1, n),
                  Wd, bd.reshape(1, d), Wmu, bmu.reshape(1, 1), n)
    return mu.reshape(-1)
